# baseline (speedup 1.0000x reference)
import jax
import jax.numpy as jnp
from jax.experimental import pallas as pl
from jax.experimental.pallas import tpu as pltpu


_C_BLOCK = 16


def _drop_max_body(x_ref, o_ref):
    x = x_ref[...]
    big = jnp.iinfo(jnp.int32).max
    iw = jax.lax.broadcasted_iota(jnp.int32, x.shape, 3)
    row_max = jnp.max(x, axis=3, keepdims=True)
    row_w = jnp.min(jnp.where(x == row_max, iw, big), axis=3, keepdims=True)
    mx = jnp.max(row_max, axis=2, keepdims=True)
    ih = jax.lax.broadcasted_iota(jnp.int32, row_max.shape, 2)
    h0 = jnp.min(jnp.where(row_max == mx, ih, big), axis=2, keepdims=True)
    w0 = jnp.min(jnp.where(ih == h0, row_w, big), axis=2, keepdims=True)
    o_ref[...] = jnp.where((ih == h0) & (iw == w0), 0.0, x)


def kernel(x):
    b, c, h, w = x.shape
    return pl.pallas_call(
        _drop_max_body,
        grid=(b, c // _C_BLOCK),
        in_specs=[pl.BlockSpec((1, _C_BLOCK, h, w), lambda i, j: (i, j, 0, 0))],
        out_specs=pl.BlockSpec((1, _C_BLOCK, h, w), lambda i, j: (i, j, 0, 0)),
        out_shape=jax.ShapeDtypeStruct((b, c, h, w), x.dtype),
        compiler_params=pltpu.CompilerParams(
            dimension_semantics=("parallel", "parallel"),
        ),
    )(x)


# speedup vs baseline: 1.0392x; 1.0392x over previous
import jax
import jax.numpy as jnp
from jax.experimental import pallas as pl
from jax.experimental.pallas import tpu as pltpu


_C_BLOCK = 24


def _drop_max_body(x_ref, o_ref):
    x = x_ref[...]
    mx = jnp.max(x, axis=(2, 3), keepdims=True)
    ih = jax.lax.broadcasted_iota(jnp.int32, x.shape, 2)
    iw = jax.lax.broadcasted_iota(jnp.int32, x.shape, 3)
    idx = ih * x.shape[3] + iw
    big = jnp.iinfo(jnp.int32).max
    first = jnp.min(jnp.where(x == mx, idx, big), axis=(2, 3), keepdims=True)
    o_ref[...] = jnp.where(idx == first, 0.0, x)


def kernel(x):
    b, c, h, w = x.shape
    return pl.pallas_call(
        _drop_max_body,
        grid=(b, c // _C_BLOCK),
        in_specs=[pl.BlockSpec((1, _C_BLOCK, h, w), lambda i, j: (i, j, 0, 0))],
        out_specs=pl.BlockSpec((1, _C_BLOCK, h, w), lambda i, j: (i, j, 0, 0)),
        out_shape=jax.ShapeDtypeStruct((b, c, h, w), x.dtype),
        compiler_params=pltpu.CompilerParams(
            dimension_semantics=("parallel", "parallel"),
        ),
    )(x)
